# baseline (speedup 1.0000x reference)
import functools

import jax
import jax.numpy as jnp
from jax import lax
from jax.experimental import pallas as pl
from jax.experimental.pallas import tpu as pltpu
from jax.experimental.pallas import tpu_sc as plsc

_B = 16384
_V = 100000
_D = 32
_DP = 128
_NCAT = 8
_NCONT = 4
_EMB_D = _NCAT * _D

_NC = 2
_NS = 16
_NW = _NC * _NS
_BPW = _B // _NW
_RC = 64
_NCHUNK = _BPW // _RC

_VBLK = 1000


def _pack4_body(t0_ref, t1_ref, t2_ref, t3_ref, out_ref):
    out_ref[:, 0 * _D:1 * _D] = t0_ref[...]
    out_ref[:, 1 * _D:2 * _D] = t1_ref[...]
    out_ref[:, 2 * _D:3 * _D] = t2_ref[...]
    out_ref[:, 3 * _D:4 * _D] = t3_ref[...]


_pack4 = pl.pallas_call(
    _pack4_body,
    grid=(_V // _VBLK,),
    in_specs=[pl.BlockSpec((_VBLK, _D), lambda i: (i, 0))] * 4,
    out_specs=pl.BlockSpec((_VBLK, _DP), lambda i: (i, 0)),
    out_shape=jax.ShapeDtypeStruct((_V, _DP), jnp.float32),
)


def _body(*refs):
    cats = refs[0:_NCAT]
    packs = refs[_NCAT:_NCAT + 2]
    out = refs[_NCAT + 2]
    idx_v, rows_v, sem = refs[_NCAT + 3:]

    wid = lax.axis_index("s") * _NC + lax.axis_index("c")
    base0 = wid * _BPW

    for ch in range(_NCHUNK):
        base = base0 + ch * _RC
        for t in range(_NCAT):
            pltpu.sync_copy(cats[t].at[pl.ds(base, _RC)], idx_v.at[t])
        gathers = [
            pltpu.async_copy(packs[t // 4].at[idx_v.at[t]], rows_v.at[t], sem)
            for t in range(_NCAT)
        ]
        for t in range(_NCAT):
            gathers[t].wait()
            pltpu.sync_copy(
                rows_v.at[t, :, pl.ds((t % 4) * _D, _D)],
                out.at[pl.ds(base, _RC), pl.ds(t * _D, _D)])


_sc_call = pl.kernel(
    _body,
    mesh=plsc.VectorSubcoreMesh(core_axis_name="c", subcore_axis_name="s"),
    out_type=jax.ShapeDtypeStruct((_B, _EMB_D), jnp.float32),
    scratch_types=[
        pltpu.VMEM((_NCAT, _RC), jnp.int32),
        pltpu.VMEM((_NCAT, _RC, _DP), jnp.float32),
        pltpu.SemaphoreType.DMA,
    ],
    compiler_params=pltpu.CompilerParams(use_tc_tiling_on_sc=False),
)


def kernel(cat_0, cat_1, cat_2, cat_3, cat_4, cat_5, cat_6, cat_7,
           table_0, table_1, table_2, table_3, table_4, table_5, table_6,
           table_7, cont_0, cont_1, cont_2, cont_3):
    cats = [c.reshape(_B).astype(jnp.int32)
            for c in (cat_0, cat_1, cat_2, cat_3, cat_4, cat_5, cat_6, cat_7)]
    pack_a = _pack4(table_0, table_1, table_2, table_3)
    pack_b = _pack4(table_4, table_5, table_6, table_7)
    emb = _sc_call(*cats, pack_a, pack_b)
    cont = jnp.stack(
        [c.astype(jnp.float32) for c in (cont_0, cont_1, cont_2, cont_3)],
        axis=-1)
    return jnp.concatenate([cont, emb], axis=-1)


# speedup vs baseline: 1.4312x; 1.4312x over previous
import jax
import jax.numpy as jnp
from jax import lax
from jax.experimental import pallas as pl
from jax.experimental.pallas import tpu as pltpu
from jax.experimental.pallas import tpu_sc as plsc

_B = 16384
_V = 100000
_D = 32
_NCAT = 8
_NCONT = 4
_EMB_D = _NCAT * _D

_NC = 2
_NS = 16
_NW = _NC * _NS
_BPW = _B // _NW
_RC = 256
_NCHUNK = _BPW // _RC


def _body(idx, *refs):
    tabs = refs[0:_NCAT]
    out = refs[_NCAT]
    idx_v, rows_v, gsem, wsem = refs[_NCAT + 1:]

    wid = lax.axis_index("s") * _NC + lax.axis_index("c")
    base0 = wid * _BPW

    for ch in range(_NCHUNK):
        base = base0 + ch * _RC
        pltpu.sync_copy(idx.at[:, pl.ds(base, _RC)], idx_v)
        gathers = [
            pltpu.async_copy(tabs[t].at[idx_v.at[t]], rows_v.at[t], gsem)
            for t in range(_NCAT)
        ]
        writes = []
        for t in range(_NCAT):
            gathers[t].wait()
            writes.append(pltpu.async_copy(
                rows_v.at[t],
                out.at[pl.ds(base, _RC), pl.ds(t * _D, _D)],
                wsem))
        for w in writes:
            w.wait()


_sc_call = pl.kernel(
    _body,
    mesh=plsc.VectorSubcoreMesh(core_axis_name="c", subcore_axis_name="s"),
    out_type=jax.ShapeDtypeStruct((_B, _EMB_D), jnp.float32),
    scratch_types=[
        pltpu.VMEM((_NCAT, _RC), jnp.int32),
        pltpu.VMEM((_NCAT, _RC, _D), jnp.float32),
        pltpu.SemaphoreType.DMA,
        pltpu.SemaphoreType.DMA,
    ],
    compiler_params=pltpu.CompilerParams(use_tc_tiling_on_sc=False),
)


def kernel(cat_0, cat_1, cat_2, cat_3, cat_4, cat_5, cat_6, cat_7,
           table_0, table_1, table_2, table_3, table_4, table_5, table_6,
           table_7, cont_0, cont_1, cont_2, cont_3):
    idx = jnp.stack(
        [c.reshape(_B).astype(jnp.int32)
         for c in (cat_0, cat_1, cat_2, cat_3, cat_4, cat_5, cat_6, cat_7)])
    emb = _sc_call(idx, table_0, table_1, table_2, table_3,
                   table_4, table_5, table_6, table_7)
    cont = jnp.stack(
        [c.astype(jnp.float32) for c in (cont_0, cont_1, cont_2, cont_3)],
        axis=-1)
    return jnp.concatenate([cont, emb], axis=-1)


# speedup vs baseline: 1.4325x; 1.0009x over previous
import jax
import jax.numpy as jnp
from jax import lax
from jax.experimental import pallas as pl
from jax.experimental.pallas import tpu as pltpu
from jax.experimental.pallas import tpu_sc as plsc

_B = 16384
_V = 100000
_D = 32
_NCAT = 8
_NCONT = 4
_EMB_D = _NCAT * _D

_NC = 2
_NS = 16
_NW = _NC * _NS
_BPW = _B // _NW
_RC = 256
_NCHUNK = _BPW // _RC


def _body(*refs):
    cats = refs[0:_NCAT]
    tabs = refs[_NCAT:2 * _NCAT]
    out = refs[2 * _NCAT]
    idx_v, rows_v, isem, gsem, wsem = refs[2 * _NCAT + 1:]

    wid = lax.axis_index("s") * _NC + lax.axis_index("c")
    base0 = wid * _BPW

    for ch in range(_NCHUNK):
        base = base0 + ch * _RC
        icopies = [
            pltpu.async_copy(cats[t].at[pl.ds(base, _RC)], idx_v.at[t], isem)
            for t in range(_NCAT)
        ]
        gathers = []
        for t in range(_NCAT):
            icopies[t].wait()
            gathers.append(pltpu.async_copy(
                tabs[t].at[idx_v.at[t]], rows_v.at[t], gsem))
        writes = []
        for t in range(_NCAT):
            gathers[t].wait()
            writes.append(pltpu.async_copy(
                rows_v.at[t],
                out.at[pl.ds(base, _RC), pl.ds(t * _D, _D)],
                wsem))
        for w in writes:
            w.wait()


_sc_call = pl.kernel(
    _body,
    mesh=plsc.VectorSubcoreMesh(core_axis_name="c", subcore_axis_name="s"),
    out_type=jax.ShapeDtypeStruct((_B, _EMB_D), jnp.float32),
    scratch_types=[
        pltpu.VMEM((_NCAT, _RC), jnp.int32),
        pltpu.VMEM((_NCAT, _RC, _D), jnp.float32),
        pltpu.SemaphoreType.DMA,
        pltpu.SemaphoreType.DMA,
        pltpu.SemaphoreType.DMA,
    ],
    compiler_params=pltpu.CompilerParams(use_tc_tiling_on_sc=False),
)


def kernel(cat_0, cat_1, cat_2, cat_3, cat_4, cat_5, cat_6, cat_7,
           table_0, table_1, table_2, table_3, table_4, table_5, table_6,
           table_7, cont_0, cont_1, cont_2, cont_3):
    cats = [c.reshape(_B).astype(jnp.int32)
            for c in (cat_0, cat_1, cat_2, cat_3, cat_4, cat_5, cat_6, cat_7)]
    emb = _sc_call(*cats, table_0, table_1, table_2, table_3,
                   table_4, table_5, table_6, table_7)
    cont = jnp.stack(
        [c.astype(jnp.float32) for c in (cont_0, cont_1, cont_2, cont_3)],
        axis=-1)
    return jnp.concatenate([cont, emb], axis=-1)


# speedup vs baseline: 1.4739x; 1.0289x over previous
import jax
import jax.numpy as jnp
from jax import lax
from jax.experimental import pallas as pl
from jax.experimental.pallas import tpu as pltpu
from jax.experimental.pallas import tpu_sc as plsc

_B = 16384
_V = 100000
_D = 32
_NCAT = 8
_NCONT = 4
_EMB_D = _NCAT * _D

_NC = 2
_NS = 16
_NW = _NC * _NS
_BPW = _B // _NW
_RC = 256
_NCHUNK = _BPW // _RC


def _body(*refs):
    cats = refs[0:_NCAT]
    tabs = refs[_NCAT:2 * _NCAT]
    out = refs[2 * _NCAT]
    idx_v, rows_v, isem, gsem, wsem = refs[2 * _NCAT + 1:]

    wid = lax.axis_index("s") * _NC + lax.axis_index("c")
    base0 = wid * _BPW

    for ch in range(_NCHUNK):
        base = base0 + ch * _RC
        icopies = [
            pltpu.async_copy(cats[t].at[pl.ds(base, _RC)], idx_v.at[t], isem)
            for t in range(_NCAT)
        ]
        gathers = []
        for t in range(_NCAT):
            icopies[t].wait()
            gathers.append(pltpu.async_copy(
                tabs[t].at[idx_v.at[t]], rows_v.at[t], gsem))
        writes = []
        for t in range(_NCAT):
            gathers[t].wait()
            writes.append(pltpu.async_copy(
                rows_v.at[t],
                out.at[pl.ds(base, _RC), pl.ds(t * _D, _D)],
                wsem))
        for w in writes:
            w.wait()


_sc_call = pl.kernel(
    _body,
    mesh=plsc.VectorSubcoreMesh(core_axis_name="c", subcore_axis_name="s"),
    out_type=jax.ShapeDtypeStruct((_B, _EMB_D), jnp.float32),
    scratch_types=[
        pltpu.VMEM((_NCAT, _RC), jnp.int32),
        pltpu.VMEM((_NCAT, _RC, _D), jnp.float32),
        pltpu.SemaphoreType.DMA,
        pltpu.SemaphoreType.DMA,
        pltpu.SemaphoreType.DMA,
    ],
    compiler_params=pltpu.CompilerParams(use_tc_tiling_on_sc=False),
)


_TC_ROWS = 2048


def _concat_body(c0, c1, c2, c3, e, o):
    o[...] = jnp.concatenate(
        [c0[...], c1[...], c2[...], c3[...], e[...]], axis=1)


_tc_concat = pl.pallas_call(
    _concat_body,
    grid=(_B // _TC_ROWS,),
    in_specs=[pl.BlockSpec((_TC_ROWS, 1), lambda i: (i, 0))] * _NCONT
    + [pl.BlockSpec((_TC_ROWS, _EMB_D), lambda i: (i, 0))],
    out_specs=pl.BlockSpec((_TC_ROWS, _NCONT + _EMB_D), lambda i: (i, 0)),
    out_shape=jax.ShapeDtypeStruct((_B, _NCONT + _EMB_D), jnp.float32),
)


def kernel(cat_0, cat_1, cat_2, cat_3, cat_4, cat_5, cat_6, cat_7,
           table_0, table_1, table_2, table_3, table_4, table_5, table_6,
           table_7, cont_0, cont_1, cont_2, cont_3):
    cats = [c.reshape(_B).astype(jnp.int32)
            for c in (cat_0, cat_1, cat_2, cat_3, cat_4, cat_5, cat_6, cat_7)]
    emb = _sc_call(*cats, table_0, table_1, table_2, table_3,
                   table_4, table_5, table_6, table_7)
    conts = [c.astype(jnp.float32).reshape(_B, 1)
             for c in (cont_0, cont_1, cont_2, cont_3)]
    return _tc_concat(*conts, emb)
